# Initial kernel scaffold; baseline (speedup 1.0000x reference)
import jax
import jax.numpy as jnp
from jax.experimental import pallas as pl


def kernel(unknown_coords, known_coords, unknown_feats, known_feats, W1, b1, g1, be1, W2, b2, g2, be2):
    raise NotImplementedError("write your pallas kernel here")


# speedup vs baseline: 28.9498x; 28.9498x over previous
import jax
import jax.numpy as jnp
from jax.experimental import pallas as pl

KNN = 3
NBLK = 512


def _k1_body(uct_ref, kc_ref, uft_ref, kft_ref, w1t_ref, b1_ref, y1_ref, s1_ref):
    uct = uct_ref[0]
    kc = kc_ref[0]
    n, m = uct.shape[0], kc.shape[1]

    su = jnp.sum(uct * uct, axis=1, keepdims=True)
    sv = jnp.sum(kc * kc, axis=0, keepdims=True)
    dot = jnp.dot(uct, kc, preferred_element_type=jnp.float32)
    d = (su + sv) - 2.0 * dot

    iota = jax.lax.broadcasted_iota(jnp.int32, (n, m), 1)
    big = jnp.int32(m)
    dd = d
    idxs, dists = [], []
    for _ in range(KNN):
        mv = jnp.min(dd, axis=1, keepdims=True)
        ii = jnp.min(jnp.where(dd == mv, iota, big), axis=1, keepdims=True)
        idxs.append(ii)
        dists.append(mv)
        dd = jnp.where(iota == ii, jnp.inf, dd)

    wk = [1.0 / (jnp.maximum(v, 0.0) + 1e-8) for v in dists]
    ws = (wk[0] + wk[1]) + wk[2]
    wn = [w / ws for w in wk]

    s_mat = (jnp.where(iota == idxs[0], wn[0], 0.0)
             + jnp.where(iota == idxs[1], wn[1], 0.0)
             + jnp.where(iota == idxs[2], wn[2], 0.0))
    inter = jnp.dot(s_mat, kft_ref[0], preferred_element_type=jnp.float32)

    x = jnp.concatenate([inter, uft_ref[0]], axis=1)
    y1 = jnp.dot(x, w1t_ref[...], preferred_element_type=jnp.float32) + b1_ref[...]
    y1_ref[0] = y1
    s1_ref[0, 0, 0:1, :] = jnp.sum(y1, axis=0, keepdims=True)
    s1_ref[0, 0, 1:2, :] = jnp.sum(y1 * y1, axis=0, keepdims=True)


def _bn_relu(y, stats, g_ref, be_ref, count):
    ssum = jnp.sum(stats[:, :, 0, :], axis=(0, 1), keepdims=False)
    ssq = jnp.sum(stats[:, :, 1, :], axis=(0, 1), keepdims=False)
    mu = (ssum / count)[None, :]
    var = (ssq / count)[None, :] - mu * mu
    rstd = jax.lax.rsqrt(var + 1e-5)
    z = (y - mu) * rstd * g_ref[...] + be_ref[...]
    return jnp.maximum(z, 0.0)


def _k2_body(y1_ref, s1_ref, g1_ref, be1_ref, w2t_ref, b2_ref, y2_ref, s2_ref,
             *, count):
    z = _bn_relu(y1_ref[0], s1_ref[...], g1_ref, be1_ref, count)
    y2 = jnp.dot(z, w2t_ref[...], preferred_element_type=jnp.float32) + b2_ref[...]
    y2_ref[0] = y2
    s2_ref[0, 0, 0:1, :] = jnp.sum(y2, axis=0, keepdims=True)
    s2_ref[0, 0, 1:2, :] = jnp.sum(y2 * y2, axis=0, keepdims=True)


def _k3_body(y2_ref, s2_ref, g2_ref, be2_ref, out_ref, *, count):
    z = _bn_relu(y2_ref[0], s2_ref[...], g2_ref, be2_ref, count)
    out_ref[0] = z.T


def kernel(unknown_coords, known_coords, unknown_feats, known_feats,
           W1, b1, g1, be1, W2, b2, g2, be2):
    B, _, n = unknown_coords.shape
    m = known_coords.shape[2]
    C1 = unknown_feats.shape[1]
    C2 = known_feats.shape[1]
    CO1 = W1.shape[0]
    CO2 = W2.shape[0]
    nb = n // NBLK
    count = float(B * n)

    uct = unknown_coords.transpose(0, 2, 1)
    uft = unknown_feats.transpose(0, 2, 1)
    kft = known_feats.transpose(0, 2, 1)
    w1t = W1.T
    w2t = W2.T
    b1r = b1.reshape(1, CO1)
    g1r = g1.reshape(1, CO1)
    be1r = be1.reshape(1, CO1)
    b2r = b2.reshape(1, CO2)
    g2r = g2.reshape(1, CO2)
    be2r = be2.reshape(1, CO2)

    grid = (B, nb)
    f32 = jnp.float32

    y1, s1 = pl.pallas_call(
        _k1_body,
        grid=grid,
        in_specs=[
            pl.BlockSpec((1, NBLK, 3), lambda b, i: (b, i, 0)),
            pl.BlockSpec((1, 3, m), lambda b, i: (b, 0, 0)),
            pl.BlockSpec((1, NBLK, C1), lambda b, i: (b, i, 0)),
            pl.BlockSpec((1, m, C2), lambda b, i: (b, 0, 0)),
            pl.BlockSpec((C2 + C1, CO1), lambda b, i: (0, 0)),
            pl.BlockSpec((1, CO1), lambda b, i: (0, 0)),
        ],
        out_specs=[
            pl.BlockSpec((1, NBLK, CO1), lambda b, i: (b, i, 0)),
            pl.BlockSpec((1, 1, 2, CO1), lambda b, i: (b, i, 0, 0)),
        ],
        out_shape=[
            jax.ShapeDtypeStruct((B, n, CO1), f32),
            jax.ShapeDtypeStruct((B, nb, 2, CO1), f32),
        ],
    )(uct, known_coords, uft, kft, w1t, b1r)

    from functools import partial
    y2, s2 = pl.pallas_call(
        partial(_k2_body, count=count),
        grid=grid,
        in_specs=[
            pl.BlockSpec((1, NBLK, CO1), lambda b, i: (b, i, 0)),
            pl.BlockSpec((B, nb, 2, CO1), lambda b, i: (0, 0, 0, 0)),
            pl.BlockSpec((1, CO1), lambda b, i: (0, 0)),
            pl.BlockSpec((1, CO1), lambda b, i: (0, 0)),
            pl.BlockSpec((CO1, CO2), lambda b, i: (0, 0)),
            pl.BlockSpec((1, CO2), lambda b, i: (0, 0)),
        ],
        out_specs=[
            pl.BlockSpec((1, NBLK, CO2), lambda b, i: (b, i, 0)),
            pl.BlockSpec((1, 1, 2, CO2), lambda b, i: (b, i, 0, 0)),
        ],
        out_shape=[
            jax.ShapeDtypeStruct((B, n, CO2), f32),
            jax.ShapeDtypeStruct((B, nb, 2, CO2), f32),
        ],
    )(y1, s1, g1r, be1r, w2t, b2r)

    out = pl.pallas_call(
        partial(_k3_body, count=count),
        grid=grid,
        in_specs=[
            pl.BlockSpec((1, NBLK, CO2), lambda b, i: (b, i, 0)),
            pl.BlockSpec((B, nb, 2, CO2), lambda b, i: (0, 0, 0, 0)),
            pl.BlockSpec((1, CO2), lambda b, i: (0, 0)),
            pl.BlockSpec((1, CO2), lambda b, i: (0, 0)),
        ],
        out_specs=pl.BlockSpec((1, CO2, NBLK), lambda b, i: (b, 0, i)),
        out_shape=jax.ShapeDtypeStruct((B, CO2, n), f32),
    )(y2, s2, g2r, be2r)
    return out


# speedup vs baseline: 30.5549x; 1.0554x over previous
import jax
import jax.numpy as jnp
from jax.experimental import pallas as pl

KNN = 3
NBLK = 512


def _dotg(a, b, dims):
    return jax.lax.dot_general(a, b, (dims, ((), ())),
                               preferred_element_type=jnp.float32)


def _k1_body(uc_ref, kc_ref, uf_ref, kf_ref, w1ta_ref, w1tb_ref, b1_ref,
             y1_ref, s1_ref):
    uc = uc_ref[0]
    kc = kc_ref[0]
    n, m = uc.shape[1], kc.shape[1]

    su = jnp.sum(uc * uc, axis=0, keepdims=True).T
    sv = jnp.sum(kc * kc, axis=0, keepdims=True)
    dot = _dotg(uc, kc, ((0,), (0,)))
    d = (su + sv) - 2.0 * dot

    iota = jax.lax.broadcasted_iota(jnp.int32, (n, m), 1)
    big = jnp.int32(m)
    dd = d
    idxs, dists = [], []
    for _ in range(KNN):
        mv = jnp.min(dd, axis=1, keepdims=True)
        ii = jnp.min(jnp.where(dd == mv, iota, big), axis=1, keepdims=True)
        idxs.append(ii)
        dists.append(mv)
        dd = jnp.where(iota == ii, jnp.inf, dd)

    wk = [1.0 / (jnp.maximum(v, 0.0) + 1e-8) for v in dists]
    ws = (wk[0] + wk[1]) + wk[2]
    wn = [w / ws for w in wk]

    s_mat = (jnp.where(iota == idxs[0], wn[0], 0.0)
             + jnp.where(iota == idxs[1], wn[1], 0.0)
             + jnp.where(iota == idxs[2], wn[2], 0.0))
    inter = _dotg(s_mat, kf_ref[0], ((1,), (1,)))

    y1 = (jnp.dot(inter, w1ta_ref[...], preferred_element_type=jnp.float32)
          + _dotg(uf_ref[0], w1tb_ref[...], ((0,), (0,)))
          + b1_ref[...])
    y1_ref[0] = y1
    s1_ref[0, 0, 0:1, :] = jnp.sum(y1, axis=0, keepdims=True)
    s1_ref[0, 0, 1:2, :] = jnp.sum(y1 * y1, axis=0, keepdims=True)


def _bn_relu(y, stats, g_ref, be_ref, count):
    ssum = jnp.sum(stats[:, :, 0, :], axis=(0, 1), keepdims=False)
    ssq = jnp.sum(stats[:, :, 1, :], axis=(0, 1), keepdims=False)
    mu = (ssum / count)[None, :]
    var = (ssq / count)[None, :] - mu * mu
    rstd = jax.lax.rsqrt(var + 1e-5)
    z = (y - mu) * rstd * g_ref[...] + be_ref[...]
    return jnp.maximum(z, 0.0)


def _k2_body(y1_ref, s1_ref, g1_ref, be1_ref, w2t_ref, b2_ref, y2_ref, s2_ref,
             *, count):
    z = _bn_relu(y1_ref[0], s1_ref[...], g1_ref, be1_ref, count)
    y2 = jnp.dot(z, w2t_ref[...], preferred_element_type=jnp.float32) + b2_ref[...]
    y2_ref[0] = y2
    s2_ref[0, 0, 0:1, :] = jnp.sum(y2, axis=0, keepdims=True)
    s2_ref[0, 0, 1:2, :] = jnp.sum(y2 * y2, axis=0, keepdims=True)


def _k3_body(y2_ref, s2_ref, g2_ref, be2_ref, out_ref, *, count):
    z = _bn_relu(y2_ref[0], s2_ref[...], g2_ref, be2_ref, count)
    out_ref[0] = z.T


def kernel(unknown_coords, known_coords, unknown_feats, known_feats,
           W1, b1, g1, be1, W2, b2, g2, be2):
    B, _, n = unknown_coords.shape
    m = known_coords.shape[2]
    C1 = unknown_feats.shape[1]
    C2 = known_feats.shape[1]
    CO1 = W1.shape[0]
    CO2 = W2.shape[0]
    nb = n // NBLK
    count = float(B * n)

    w1ta = W1[:, :C2].T
    w1tb = W1[:, C2:].T
    w2t = W2.T
    b1r = b1.reshape(1, CO1)
    g1r = g1.reshape(1, CO1)
    be1r = be1.reshape(1, CO1)
    b2r = b2.reshape(1, CO2)
    g2r = g2.reshape(1, CO2)
    be2r = be2.reshape(1, CO2)

    grid = (B, nb)
    f32 = jnp.float32

    y1, s1 = pl.pallas_call(
        _k1_body,
        grid=grid,
        in_specs=[
            pl.BlockSpec((1, 3, NBLK), lambda b, i: (b, 0, i)),
            pl.BlockSpec((1, 3, m), lambda b, i: (b, 0, 0)),
            pl.BlockSpec((1, C1, NBLK), lambda b, i: (b, 0, i)),
            pl.BlockSpec((1, C2, m), lambda b, i: (b, 0, 0)),
            pl.BlockSpec((C2, CO1), lambda b, i: (0, 0)),
            pl.BlockSpec((C1, CO1), lambda b, i: (0, 0)),
            pl.BlockSpec((1, CO1), lambda b, i: (0, 0)),
        ],
        out_specs=[
            pl.BlockSpec((1, NBLK, CO1), lambda b, i: (b, i, 0)),
            pl.BlockSpec((1, 1, 2, CO1), lambda b, i: (b, i, 0, 0)),
        ],
        out_shape=[
            jax.ShapeDtypeStruct((B, n, CO1), f32),
            jax.ShapeDtypeStruct((B, nb, 2, CO1), f32),
        ],
    )(unknown_coords, known_coords, unknown_feats, known_feats,
      w1ta, w1tb, b1r)

    from functools import partial
    y2, s2 = pl.pallas_call(
        partial(_k2_body, count=count),
        grid=grid,
        in_specs=[
            pl.BlockSpec((1, NBLK, CO1), lambda b, i: (b, i, 0)),
            pl.BlockSpec((B, nb, 2, CO1), lambda b, i: (0, 0, 0, 0)),
            pl.BlockSpec((1, CO1), lambda b, i: (0, 0)),
            pl.BlockSpec((1, CO1), lambda b, i: (0, 0)),
            pl.BlockSpec((CO1, CO2), lambda b, i: (0, 0)),
            pl.BlockSpec((1, CO2), lambda b, i: (0, 0)),
        ],
        out_specs=[
            pl.BlockSpec((1, NBLK, CO2), lambda b, i: (b, i, 0)),
            pl.BlockSpec((1, 1, 2, CO2), lambda b, i: (b, i, 0, 0)),
        ],
        out_shape=[
            jax.ShapeDtypeStruct((B, n, CO2), f32),
            jax.ShapeDtypeStruct((B, nb, 2, CO2), f32),
        ],
    )(y1, s1, g1r, be1r, w2t, b2r)

    out = pl.pallas_call(
        partial(_k3_body, count=count),
        grid=grid,
        in_specs=[
            pl.BlockSpec((1, NBLK, CO2), lambda b, i: (b, i, 0)),
            pl.BlockSpec((B, nb, 2, CO2), lambda b, i: (0, 0, 0, 0)),
            pl.BlockSpec((1, CO2), lambda b, i: (0, 0)),
            pl.BlockSpec((1, CO2), lambda b, i: (0, 0)),
        ],
        out_specs=pl.BlockSpec((1, CO2, NBLK), lambda b, i: (b, 0, i)),
        out_shape=jax.ShapeDtypeStruct((B, CO2, n), f32),
    )(y2, s2, g2r, be2r)
    return out


# speedup vs baseline: 40.5236x; 1.3263x over previous
import jax
import jax.numpy as jnp
from jax.experimental import pallas as pl

KNN = 3
NBLK = 512


def _dotg(a, b, dims):
    return jax.lax.dot_general(a, b, (dims, ((), ())),
                               preferred_element_type=jnp.float32)


def _k1_body(uc_ref, kc_ref, uf_ref, kf_ref, w1ta_ref, w1tb_ref, b1_ref,
             y1_ref, s1_ref):
    uc = uc_ref[0]
    kc = kc_ref[0]
    n, m = uc.shape[1], kc.shape[1]

    su = jnp.sum(uc * uc, axis=0, keepdims=True).T
    sv = jnp.sum(kc * kc, axis=0, keepdims=True)
    dot = _dotg(uc, kc, ((0,), (0,)))
    d = (su + sv) - 2.0 * dot

    v1 = jnp.min(d, axis=1, keepdims=True)
    d2 = jnp.where(d <= v1, jnp.inf, d)
    v2 = jnp.min(d2, axis=1, keepdims=True)
    d3 = jnp.where(d2 <= v2, jnp.inf, d2)
    v3 = jnp.min(d3, axis=1, keepdims=True)
    dists = [v1, v2, v3]

    wk = [1.0 / (jnp.maximum(v, 0.0) + 1e-8) for v in dists]
    ws = (wk[0] + wk[1]) + wk[2]
    wn = [w / ws for w in wk]

    s_mat = (jnp.where(d == v1, wn[0], 0.0)
             + jnp.where(d == v2, wn[1], 0.0)
             + jnp.where(d == v3, wn[2], 0.0))
    inter = _dotg(s_mat, kf_ref[0], ((1,), (1,)))

    y1 = (jnp.dot(inter, w1ta_ref[...], preferred_element_type=jnp.float32)
          + _dotg(uf_ref[0], w1tb_ref[...], ((0,), (0,)))
          + b1_ref[...])
    y1_ref[0] = y1
    s1_ref[0, 0, 0:1, :] = jnp.sum(y1, axis=0, keepdims=True)
    s1_ref[0, 0, 1:2, :] = jnp.sum(y1 * y1, axis=0, keepdims=True)


def _bn_relu(y, stats, g_ref, be_ref, count):
    ssum = jnp.sum(stats[:, :, 0, :], axis=(0, 1), keepdims=False)
    ssq = jnp.sum(stats[:, :, 1, :], axis=(0, 1), keepdims=False)
    mu = (ssum / count)[None, :]
    var = (ssq / count)[None, :] - mu * mu
    rstd = jax.lax.rsqrt(var + 1e-5)
    z = (y - mu) * rstd * g_ref[...] + be_ref[...]
    return jnp.maximum(z, 0.0)


def _k2_body(y1_ref, s1_ref, g1_ref, be1_ref, w2t_ref, b2_ref, y2_ref, s2_ref,
             *, count):
    z = _bn_relu(y1_ref[0], s1_ref[...], g1_ref, be1_ref, count)
    y2 = jnp.dot(z, w2t_ref[...], preferred_element_type=jnp.float32) + b2_ref[...]
    y2_ref[0] = y2
    s2_ref[0, 0, 0:1, :] = jnp.sum(y2, axis=0, keepdims=True)
    s2_ref[0, 0, 1:2, :] = jnp.sum(y2 * y2, axis=0, keepdims=True)


def _k3_body(y2_ref, s2_ref, g2_ref, be2_ref, out_ref, *, count):
    z = _bn_relu(y2_ref[0], s2_ref[...], g2_ref, be2_ref, count)
    out_ref[0] = z.T


def kernel(unknown_coords, known_coords, unknown_feats, known_feats,
           W1, b1, g1, be1, W2, b2, g2, be2):
    B, _, n = unknown_coords.shape
    m = known_coords.shape[2]
    C1 = unknown_feats.shape[1]
    C2 = known_feats.shape[1]
    CO1 = W1.shape[0]
    CO2 = W2.shape[0]
    nb = n // NBLK
    count = float(B * n)

    w1ta = W1[:, :C2].T
    w1tb = W1[:, C2:].T
    w2t = W2.T
    b1r = b1.reshape(1, CO1)
    g1r = g1.reshape(1, CO1)
    be1r = be1.reshape(1, CO1)
    b2r = b2.reshape(1, CO2)
    g2r = g2.reshape(1, CO2)
    be2r = be2.reshape(1, CO2)

    grid = (B, nb)
    f32 = jnp.float32

    y1, s1 = pl.pallas_call(
        _k1_body,
        grid=grid,
        in_specs=[
            pl.BlockSpec((1, 3, NBLK), lambda b, i: (b, 0, i)),
            pl.BlockSpec((1, 3, m), lambda b, i: (b, 0, 0)),
            pl.BlockSpec((1, C1, NBLK), lambda b, i: (b, 0, i)),
            pl.BlockSpec((1, C2, m), lambda b, i: (b, 0, 0)),
            pl.BlockSpec((C2, CO1), lambda b, i: (0, 0)),
            pl.BlockSpec((C1, CO1), lambda b, i: (0, 0)),
            pl.BlockSpec((1, CO1), lambda b, i: (0, 0)),
        ],
        out_specs=[
            pl.BlockSpec((1, NBLK, CO1), lambda b, i: (b, i, 0)),
            pl.BlockSpec((1, 1, 2, CO1), lambda b, i: (b, i, 0, 0)),
        ],
        out_shape=[
            jax.ShapeDtypeStruct((B, n, CO1), f32),
            jax.ShapeDtypeStruct((B, nb, 2, CO1), f32),
        ],
    )(unknown_coords, known_coords, unknown_feats, known_feats,
      w1ta, w1tb, b1r)

    from functools import partial
    y2, s2 = pl.pallas_call(
        partial(_k2_body, count=count),
        grid=grid,
        in_specs=[
            pl.BlockSpec((1, NBLK, CO1), lambda b, i: (b, i, 0)),
            pl.BlockSpec((B, nb, 2, CO1), lambda b, i: (0, 0, 0, 0)),
            pl.BlockSpec((1, CO1), lambda b, i: (0, 0)),
            pl.BlockSpec((1, CO1), lambda b, i: (0, 0)),
            pl.BlockSpec((CO1, CO2), lambda b, i: (0, 0)),
            pl.BlockSpec((1, CO2), lambda b, i: (0, 0)),
        ],
        out_specs=[
            pl.BlockSpec((1, NBLK, CO2), lambda b, i: (b, i, 0)),
            pl.BlockSpec((1, 1, 2, CO2), lambda b, i: (b, i, 0, 0)),
        ],
        out_shape=[
            jax.ShapeDtypeStruct((B, n, CO2), f32),
            jax.ShapeDtypeStruct((B, nb, 2, CO2), f32),
        ],
    )(y1, s1, g1r, be1r, w2t, b2r)

    out = pl.pallas_call(
        partial(_k3_body, count=count),
        grid=grid,
        in_specs=[
            pl.BlockSpec((1, NBLK, CO2), lambda b, i: (b, i, 0)),
            pl.BlockSpec((B, nb, 2, CO2), lambda b, i: (0, 0, 0, 0)),
            pl.BlockSpec((1, CO2), lambda b, i: (0, 0)),
            pl.BlockSpec((1, CO2), lambda b, i: (0, 0)),
        ],
        out_specs=pl.BlockSpec((1, CO2, NBLK), lambda b, i: (b, 0, i)),
        out_shape=jax.ShapeDtypeStruct((B, CO2, n), f32),
    )(y2, s2, g2r, be2r)
    return out


# speedup vs baseline: 40.6543x; 1.0032x over previous
import jax
import jax.numpy as jnp
from jax.experimental import pallas as pl

KNN = 3
NBLK = 512


def _dotg(a, b, dims):
    return jax.lax.dot_general(a, b, (dims, ((), ())),
                               preferred_element_type=jnp.float32)


def _k1_body(uc_ref, kc_ref, uf_ref, kf_ref, w1ta_ref, w1tb_ref, b1_ref,
             y1_ref, s1_ref):
    uc = uc_ref[0]
    kc = kc_ref[0]
    n, m = uc.shape[1], kc.shape[1]

    su = jnp.sum(uc * uc, axis=0, keepdims=True).T
    sv = jnp.sum(kc * kc, axis=0, keepdims=True)
    dot = _dotg(uc, kc, ((0,), (0,)))
    d = (su + sv) - 2.0 * dot

    v1 = jnp.min(d, axis=1, keepdims=True)
    d2 = jnp.where(d == v1, jnp.inf, d)
    v2 = jnp.min(d2, axis=1, keepdims=True)
    d3 = jnp.where(d2 == v2, jnp.inf, d2)
    v3 = jnp.min(d3, axis=1, keepdims=True)
    dists = [v1, v2, v3]

    wk = [1.0 / (jnp.maximum(v, 0.0) + 1e-8) for v in dists]
    ws = (wk[0] + wk[1]) + wk[2]
    wn = [w / ws for w in wk]

    s_mat = jnp.where(d == v1, wn[0],
                      jnp.where(d == v2, wn[1],
                                jnp.where(d3 == v3, wn[2], 0.0)))
    inter = _dotg(s_mat, kf_ref[0], ((1,), (1,)))

    y1 = (jnp.dot(inter, w1ta_ref[...], preferred_element_type=jnp.float32)
          + _dotg(uf_ref[0], w1tb_ref[...], ((0,), (0,)))
          + b1_ref[...])
    y1_ref[0] = y1
    s1_ref[0, 0, 0:1, :] = jnp.sum(y1, axis=0, keepdims=True)
    s1_ref[0, 0, 1:2, :] = jnp.sum(y1 * y1, axis=0, keepdims=True)


def _bn_relu(y, stats, g_ref, be_ref, count):
    ssum = jnp.sum(stats[:, :, 0, :], axis=(0, 1), keepdims=False)
    ssq = jnp.sum(stats[:, :, 1, :], axis=(0, 1), keepdims=False)
    mu = (ssum / count)[None, :]
    var = (ssq / count)[None, :] - mu * mu
    rstd = jax.lax.rsqrt(var + 1e-5)
    z = (y - mu) * rstd * g_ref[...] + be_ref[...]
    return jnp.maximum(z, 0.0)


def _k2_body(y1_ref, s1_ref, g1_ref, be1_ref, w2t_ref, b2_ref, y2_ref, s2_ref,
             *, count):
    z = _bn_relu(y1_ref[0], s1_ref[...], g1_ref, be1_ref, count)
    y2 = jnp.dot(z, w2t_ref[...], preferred_element_type=jnp.float32) + b2_ref[...]
    y2_ref[0] = y2
    s2_ref[0, 0, 0:1, :] = jnp.sum(y2, axis=0, keepdims=True)
    s2_ref[0, 0, 1:2, :] = jnp.sum(y2 * y2, axis=0, keepdims=True)


def _k3_body(y2_ref, s2_ref, g2_ref, be2_ref, out_ref, *, count):
    z = _bn_relu(y2_ref[0], s2_ref[...], g2_ref, be2_ref, count)
    out_ref[0] = z.T


def kernel(unknown_coords, known_coords, unknown_feats, known_feats,
           W1, b1, g1, be1, W2, b2, g2, be2):
    B, _, n = unknown_coords.shape
    m = known_coords.shape[2]
    C1 = unknown_feats.shape[1]
    C2 = known_feats.shape[1]
    CO1 = W1.shape[0]
    CO2 = W2.shape[0]
    nb = n // NBLK
    count = float(B * n)

    w1ta = W1[:, :C2].T
    w1tb = W1[:, C2:].T
    w2t = W2.T
    b1r = b1.reshape(1, CO1)
    g1r = g1.reshape(1, CO1)
    be1r = be1.reshape(1, CO1)
    b2r = b2.reshape(1, CO2)
    g2r = g2.reshape(1, CO2)
    be2r = be2.reshape(1, CO2)

    grid = (B, nb)
    f32 = jnp.float32

    y1, s1 = pl.pallas_call(
        _k1_body,
        grid=grid,
        in_specs=[
            pl.BlockSpec((1, 3, NBLK), lambda b, i: (b, 0, i)),
            pl.BlockSpec((1, 3, m), lambda b, i: (b, 0, 0)),
            pl.BlockSpec((1, C1, NBLK), lambda b, i: (b, 0, i)),
            pl.BlockSpec((1, C2, m), lambda b, i: (b, 0, 0)),
            pl.BlockSpec((C2, CO1), lambda b, i: (0, 0)),
            pl.BlockSpec((C1, CO1), lambda b, i: (0, 0)),
            pl.BlockSpec((1, CO1), lambda b, i: (0, 0)),
        ],
        out_specs=[
            pl.BlockSpec((1, NBLK, CO1), lambda b, i: (b, i, 0)),
            pl.BlockSpec((1, 1, 2, CO1), lambda b, i: (b, i, 0, 0)),
        ],
        out_shape=[
            jax.ShapeDtypeStruct((B, n, CO1), f32),
            jax.ShapeDtypeStruct((B, nb, 2, CO1), f32),
        ],
    )(unknown_coords, known_coords, unknown_feats, known_feats,
      w1ta, w1tb, b1r)

    from functools import partial
    y2, s2 = pl.pallas_call(
        partial(_k2_body, count=count),
        grid=grid,
        in_specs=[
            pl.BlockSpec((1, NBLK, CO1), lambda b, i: (b, i, 0)),
            pl.BlockSpec((B, nb, 2, CO1), lambda b, i: (0, 0, 0, 0)),
            pl.BlockSpec((1, CO1), lambda b, i: (0, 0)),
            pl.BlockSpec((1, CO1), lambda b, i: (0, 0)),
            pl.BlockSpec((CO1, CO2), lambda b, i: (0, 0)),
            pl.BlockSpec((1, CO2), lambda b, i: (0, 0)),
        ],
        out_specs=[
            pl.BlockSpec((1, NBLK, CO2), lambda b, i: (b, i, 0)),
            pl.BlockSpec((1, 1, 2, CO2), lambda b, i: (b, i, 0, 0)),
        ],
        out_shape=[
            jax.ShapeDtypeStruct((B, n, CO2), f32),
            jax.ShapeDtypeStruct((B, nb, 2, CO2), f32),
        ],
    )(y1, s1, g1r, be1r, w2t, b2r)

    out = pl.pallas_call(
        partial(_k3_body, count=count),
        grid=grid,
        in_specs=[
            pl.BlockSpec((1, NBLK, CO2), lambda b, i: (b, i, 0)),
            pl.BlockSpec((B, nb, 2, CO2), lambda b, i: (0, 0, 0, 0)),
            pl.BlockSpec((1, CO2), lambda b, i: (0, 0)),
            pl.BlockSpec((1, CO2), lambda b, i: (0, 0)),
        ],
        out_specs=pl.BlockSpec((1, CO2, NBLK), lambda b, i: (b, 0, i)),
        out_shape=jax.ShapeDtypeStruct((B, CO2, n), f32),
    )(y2, s2, g2r, be2r)
    return out


# speedup vs baseline: 47.3465x; 1.1646x over previous
import jax
import jax.numpy as jnp
from jax.experimental import pallas as pl

KNN = 3
NBLK = 1024


def _dotg(a, b, dims):
    return jax.lax.dot_general(a, b, (dims, ((), ())),
                               preferred_element_type=jnp.float32)


def _k1_body(uc_ref, kcs_ref, uf_ref, kf_ref, w1ta_ref, w1tb_ref, b1_ref,
             y1_ref, s1_ref):
    uc = uc_ref[0]
    kcs = kcs_ref[0]
    n, m = uc.shape[1], kcs.shape[1]

    su = jnp.sum(uc * uc, axis=0, keepdims=True).T
    sv = 0.25 * jnp.sum(kcs * kcs, axis=0, keepdims=True)
    ndot2 = _dotg(uc, kcs, ((0,), (0,)))
    d = (su + sv) + ndot2

    v1 = jnp.min(d, axis=1, keepdims=True)
    d2 = jnp.where(d == v1, jnp.inf, d)
    v2 = jnp.min(d2, axis=1, keepdims=True)
    d3 = jnp.where(d2 == v2, jnp.inf, d2)
    v3 = jnp.min(d3, axis=1, keepdims=True)
    dists = [v1, v2, v3]

    wk = [1.0 / (jnp.maximum(v, 0.0) + 1e-8) for v in dists]
    ws = (wk[0] + wk[1]) + wk[2]
    wn = [w / ws for w in wk]

    s_mat = jnp.where(d == v1, wn[0],
                      jnp.where(d == v2, wn[1],
                                jnp.where(d3 == v3, wn[2], 0.0)))
    inter = _dotg(s_mat, kf_ref[0], ((1,), (1,)))

    y1 = (jnp.dot(inter, w1ta_ref[...], preferred_element_type=jnp.float32)
          + _dotg(uf_ref[0], w1tb_ref[...], ((0,), (0,)))
          + b1_ref[...])
    y1_ref[0] = y1
    s1_ref[0, 0, 0:1, :] = jnp.sum(y1, axis=0, keepdims=True)
    s1_ref[0, 0, 1:2, :] = jnp.sum(y1 * y1, axis=0, keepdims=True)


def _bn_relu(y, stats, g_ref, be_ref, count):
    ssum = jnp.sum(stats[:, :, 0, :], axis=(0, 1), keepdims=False)
    ssq = jnp.sum(stats[:, :, 1, :], axis=(0, 1), keepdims=False)
    mu = (ssum / count)[None, :]
    var = (ssq / count)[None, :] - mu * mu
    rstd = jax.lax.rsqrt(var + 1e-5)
    z = (y - mu) * rstd * g_ref[...] + be_ref[...]
    return jnp.maximum(z, 0.0)


def _k2_body(y1_ref, s1_ref, g1_ref, be1_ref, w2t_ref, b2_ref, y2_ref, s2_ref,
             *, count):
    z = _bn_relu(y1_ref[0], s1_ref[...], g1_ref, be1_ref, count)
    y2 = jnp.dot(z, w2t_ref[...], preferred_element_type=jnp.float32) + b2_ref[...]
    y2_ref[0] = y2
    s2_ref[0, 0, 0:1, :] = jnp.sum(y2, axis=0, keepdims=True)
    s2_ref[0, 0, 1:2, :] = jnp.sum(y2 * y2, axis=0, keepdims=True)


def _k3_body(y2_ref, s2_ref, g2_ref, be2_ref, out_ref, *, count):
    z = _bn_relu(y2_ref[0], s2_ref[...], g2_ref, be2_ref, count)
    out_ref[0] = z.T


def kernel(unknown_coords, known_coords, unknown_feats, known_feats,
           W1, b1, g1, be1, W2, b2, g2, be2):
    B, _, n = unknown_coords.shape
    m = known_coords.shape[2]
    C1 = unknown_feats.shape[1]
    C2 = known_feats.shape[1]
    CO1 = W1.shape[0]
    CO2 = W2.shape[0]
    nb = n // NBLK
    count = float(B * n)

    w1ta = W1[:, :C2].T
    w1tb = W1[:, C2:].T
    w2t = W2.T
    b1r = b1.reshape(1, CO1)
    g1r = g1.reshape(1, CO1)
    be1r = be1.reshape(1, CO1)
    b2r = b2.reshape(1, CO2)
    g2r = g2.reshape(1, CO2)
    be2r = be2.reshape(1, CO2)

    grid = (B, nb)
    f32 = jnp.float32

    y1, s1 = pl.pallas_call(
        _k1_body,
        grid=grid,
        in_specs=[
            pl.BlockSpec((1, 3, NBLK), lambda b, i: (b, 0, i)),
            pl.BlockSpec((1, 3, m), lambda b, i: (b, 0, 0)),
            pl.BlockSpec((1, C1, NBLK), lambda b, i: (b, 0, i)),
            pl.BlockSpec((1, C2, m), lambda b, i: (b, 0, 0)),
            pl.BlockSpec((C2, CO1), lambda b, i: (0, 0)),
            pl.BlockSpec((C1, CO1), lambda b, i: (0, 0)),
            pl.BlockSpec((1, CO1), lambda b, i: (0, 0)),
        ],
        out_specs=[
            pl.BlockSpec((1, NBLK, CO1), lambda b, i: (b, i, 0)),
            pl.BlockSpec((1, 1, 2, CO1), lambda b, i: (b, i, 0, 0)),
        ],
        out_shape=[
            jax.ShapeDtypeStruct((B, n, CO1), f32),
            jax.ShapeDtypeStruct((B, nb, 2, CO1), f32),
        ],
    )(unknown_coords, -2.0 * known_coords, unknown_feats, known_feats,
      w1ta, w1tb, b1r)

    from functools import partial
    y2, s2 = pl.pallas_call(
        partial(_k2_body, count=count),
        grid=grid,
        in_specs=[
            pl.BlockSpec((1, NBLK, CO1), lambda b, i: (b, i, 0)),
            pl.BlockSpec((B, nb, 2, CO1), lambda b, i: (0, 0, 0, 0)),
            pl.BlockSpec((1, CO1), lambda b, i: (0, 0)),
            pl.BlockSpec((1, CO1), lambda b, i: (0, 0)),
            pl.BlockSpec((CO1, CO2), lambda b, i: (0, 0)),
            pl.BlockSpec((1, CO2), lambda b, i: (0, 0)),
        ],
        out_specs=[
            pl.BlockSpec((1, NBLK, CO2), lambda b, i: (b, i, 0)),
            pl.BlockSpec((1, 1, 2, CO2), lambda b, i: (b, i, 0, 0)),
        ],
        out_shape=[
            jax.ShapeDtypeStruct((B, n, CO2), f32),
            jax.ShapeDtypeStruct((B, nb, 2, CO2), f32),
        ],
    )(y1, s1, g1r, be1r, w2t, b2r)

    out = pl.pallas_call(
        partial(_k3_body, count=count),
        grid=grid,
        in_specs=[
            pl.BlockSpec((1, NBLK, CO2), lambda b, i: (b, i, 0)),
            pl.BlockSpec((B, nb, 2, CO2), lambda b, i: (0, 0, 0, 0)),
            pl.BlockSpec((1, CO2), lambda b, i: (0, 0)),
            pl.BlockSpec((1, CO2), lambda b, i: (0, 0)),
        ],
        out_specs=pl.BlockSpec((1, CO2, NBLK), lambda b, i: (b, 0, i)),
        out_shape=jax.ShapeDtypeStruct((B, CO2, n), f32),
    )(y2, s2, g2r, be2r)
    return out
